# Initial kernel scaffold; baseline (speedup 1.0000x reference)
import functools

import jax
import jax.numpy as jnp
from jax import lax
from jax.experimental import pallas as pl
from jax.experimental.pallas import tpu as pltpu
from jax.experimental.pallas import tpu_sc as plsc

NUM_EMB = 1_000_000
DIM = 64
BATCH = 16384
HIST = 50
TOTAL = BATCH * HIST

NUM_CORES = 2
NUM_SUBCORES = 16
NW = NUM_CORES * NUM_SUBCORES
ROWS_PER_W = TOTAL // NW

IDX_MINOR = 128
CHUNK = 512
SUB = CHUNK // IDX_MINOR
NCHUNK = ROWS_PER_W // CHUNK

_mesh = plsc.VectorSubcoreMesh(core_axis_name="c", subcore_axis_name="s")


@functools.partial(
    pl.kernel,
    mesh=_mesh,
    out_type=jax.ShapeDtypeStruct((TOTAL, DIM), jnp.float32),
    scratch_types=[
        pltpu.VMEM((SUB, IDX_MINOR), jnp.int32),
        pltpu.VMEM((CHUNK, DIM), jnp.float32),
        pltpu.SemaphoreType.DMA,
    ],
)
def _embed_sc(idx_hbm, table_hbm, out_hbm, idx_v, rows_v, sem):
    wid = lax.axis_index("s") * NUM_CORES + lax.axis_index("c")
    base_row = wid * ROWS_PER_W
    base_blk = wid * (ROWS_PER_W // IDX_MINOR)

    def body(g, carry):
        blk = base_blk + g * SUB
        pltpu.sync_copy(idx_hbm.at[pl.ds(blk, SUB)], idx_v)
        copies = []
        for j in range(SUB):
            copies.append(
                pltpu.async_copy(
                    table_hbm.at[idx_v.at[j]],
                    rows_v.at[pl.ds(j * IDX_MINOR, IDX_MINOR)],
                    sem,
                )
            )
        for c in copies:
            c.wait()
        off = base_row + g * CHUNK
        pltpu.sync_copy(rows_v, out_hbm.at[pl.ds(off, CHUNK)])
        return carry

    lax.fori_loop(0, NCHUNK, body, 0)


def kernel(token_ids, weight):
    idx = token_ids.reshape(TOTAL // IDX_MINOR, IDX_MINOR)
    out = _embed_sc(idx, weight)
    return out.reshape(BATCH, HIST, DIM)


# speedup vs baseline: 1.7970x; 1.7970x over previous
import functools

import jax
import jax.numpy as jnp
from jax import lax
from jax.experimental import pallas as pl
from jax.experimental.pallas import tpu as pltpu
from jax.experimental.pallas import tpu_sc as plsc

NUM_EMB = 1_000_000
DIM = 64
BATCH = 16384
HIST = 50
TOTAL = BATCH * HIST

NUM_CORES = 2
NUM_SUBCORES = 16
NW = NUM_CORES * NUM_SUBCORES
ROWS_PER_W = TOTAL // NW

IDX_MINOR = 128
CHUNK = 512
SUB = CHUNK // IDX_MINOR
NCHUNK = ROWS_PER_W // CHUNK

_mesh = plsc.VectorSubcoreMesh(core_axis_name="c", subcore_axis_name="s")


@functools.partial(
    pl.kernel,
    mesh=_mesh,
    out_type=jax.ShapeDtypeStruct((TOTAL, DIM), jnp.float32),
    scratch_types=[
        pltpu.VMEM((SUB, IDX_MINOR), jnp.int32),
        pltpu.VMEM((CHUNK, DIM), jnp.float32),
        pltpu.SemaphoreType.DMA,
    ],
    compiler_params=pltpu.CompilerParams(use_tc_tiling_on_sc=False),
)
def _embed_sc(idx_hbm, table_hbm, out_hbm, idx_v, rows_v, sem):
    wid = lax.axis_index("s") * NUM_CORES + lax.axis_index("c")
    base_row = wid * ROWS_PER_W
    base_blk = wid * (ROWS_PER_W // IDX_MINOR)

    def body(g, carry):
        blk = base_blk + g * SUB
        pltpu.sync_copy(idx_hbm.at[pl.ds(blk, SUB)], idx_v)
        copies = []
        for j in range(SUB):
            copies.append(
                pltpu.async_copy(
                    table_hbm.at[idx_v.at[j]],
                    rows_v.at[pl.ds(j * IDX_MINOR, IDX_MINOR)],
                    sem,
                )
            )
        for c in copies:
            c.wait()
        off = base_row + g * CHUNK
        pltpu.sync_copy(rows_v, out_hbm.at[pl.ds(off, CHUNK)])
        return carry

    lax.fori_loop(0, NCHUNK, body, 0)


def kernel(token_ids, weight):
    idx = token_ids.reshape(TOTAL // IDX_MINOR, IDX_MINOR)
    out = _embed_sc(idx, weight)
    return out.reshape(BATCH, HIST, DIM)


# speedup vs baseline: 1.8721x; 1.0418x over previous
import functools

import jax
import jax.numpy as jnp
from jax import lax
from jax.experimental import pallas as pl
from jax.experimental.pallas import tpu as pltpu
from jax.experimental.pallas import tpu_sc as plsc

NUM_EMB = 1_000_000
DIM = 64
BATCH = 16384
HIST = 50
TOTAL = BATCH * HIST

NUM_CORES = 2
NUM_SUBCORES = 16
NW = NUM_CORES * NUM_SUBCORES
ROWS_PER_W = TOTAL // NW

IDX_MINOR = 128
CHUNK = 256
SUB = CHUNK // IDX_MINOR
NCHUNK = ROWS_PER_W // CHUNK
NBUF = 4
NOUTER = NCHUNK // NBUF
IDX_ROWS = ROWS_PER_W // IDX_MINOR

_mesh = plsc.VectorSubcoreMesh(core_axis_name="c", subcore_axis_name="s")


@functools.partial(
    pl.kernel,
    mesh=_mesh,
    out_type=jax.ShapeDtypeStruct((TOTAL, DIM), jnp.float32),
    scratch_types=[
        pltpu.VMEM((IDX_ROWS, IDX_MINOR), jnp.int32),
        pltpu.VMEM((CHUNK, DIM), jnp.float32),
        pltpu.VMEM((CHUNK, DIM), jnp.float32),
        pltpu.VMEM((CHUNK, DIM), jnp.float32),
        pltpu.VMEM((CHUNK, DIM), jnp.float32),
        pltpu.SemaphoreType.DMA,
        pltpu.SemaphoreType.DMA,
        pltpu.SemaphoreType.DMA,
        pltpu.SemaphoreType.DMA,
        pltpu.SemaphoreType.DMA,
        pltpu.SemaphoreType.DMA,
        pltpu.SemaphoreType.DMA,
        pltpu.SemaphoreType.DMA,
    ],
    compiler_params=pltpu.CompilerParams(use_tc_tiling_on_sc=False),
)
def _embed_sc(idx_hbm, table_hbm, out_hbm,
              idx_v, r0, r1, r2, r3,
              g0, g1, g2, g3, o0, o1, o2, o3):
    rows = (r0, r1, r2, r3)
    gsem = (g0, g1, g2, g3)
    osem = (o0, o1, o2, o3)

    wid = lax.axis_index("s") * NUM_CORES + lax.axis_index("c")
    base_row = wid * ROWS_PER_W
    base_blk = wid * IDX_ROWS

    pltpu.sync_copy(idx_hbm.at[pl.ds(base_blk, IDX_ROWS)], idx_v)

    def issue_gather(c, slot):
        for jj in range(SUB):
            pltpu.async_copy(
                table_hbm.at[idx_v.at[c * SUB + jj]],
                rows[slot].at[pl.ds(jj * IDX_MINOR, IDX_MINOR)],
                gsem[slot],
            )

    def wait_gather(slot):
        pltpu.make_async_copy(
            out_hbm.at[pl.ds(0, CHUNK)], rows[slot], gsem[slot]
        ).wait()

    def issue_store(c, slot):
        pltpu.async_copy(
            rows[slot], out_hbm.at[pl.ds(base_row + c * CHUNK, CHUNK)],
            osem[slot],
        )

    def wait_store(slot):
        pltpu.make_async_copy(
            rows[slot], out_hbm.at[pl.ds(base_row, CHUNK)], osem[slot]
        ).wait()

    issue_gather(0, 0)
    issue_gather(1, 1)
    wait_gather(0); issue_store(0, 0); issue_gather(2, 2)
    wait_gather(1); issue_store(1, 1); issue_gather(3, 3)
    wait_gather(2); issue_store(2, 2); wait_store(0); issue_gather(4, 0)
    wait_gather(3); issue_store(3, 3); wait_store(1); issue_gather(5, 1)

    def body(t, carry):
        c0 = t * NBUF
        for j in range(NBUF):
            c = c0 + j
            wait_gather(j)
            issue_store(c, j)
            wait_store((j + 2) % NBUF)
            issue_gather(c + 2, (j + 2) % NBUF)
        return carry

    lax.fori_loop(1, NOUTER - 1, body, 0)

    cL = (NOUTER - 1) * NBUF
    wait_gather(0); issue_store(cL + 0, 0); wait_store(2); issue_gather(cL + 2, 2)
    wait_gather(1); issue_store(cL + 1, 1); wait_store(3); issue_gather(cL + 3, 3)
    wait_gather(2); issue_store(cL + 2, 2); wait_store(0)
    wait_gather(3); issue_store(cL + 3, 3); wait_store(1)
    wait_store(2)
    wait_store(3)


def kernel(token_ids, weight):
    idx = token_ids.reshape(TOTAL // IDX_MINOR, IDX_MINOR)
    out = _embed_sc(idx, weight)
    return out.reshape(BATCH, HIST, DIM)


# speedup vs baseline: 1.9561x; 1.0449x over previous
import functools

import jax
import jax.numpy as jnp
from jax import lax
from jax.experimental import pallas as pl
from jax.experimental.pallas import tpu as pltpu
from jax.experimental.pallas import tpu_sc as plsc

NUM_EMB = 1_000_000
DIM = 64
BATCH = 16384
HIST = 50
TOTAL = BATCH * HIST

NUM_CORES = 2
NUM_SUBCORES = 16
NW = NUM_CORES * NUM_SUBCORES
ROWS_PER_W = TOTAL // NW

IDX_MINOR = 128
CHUNK = 256
SUB = CHUNK // IDX_MINOR
NCHUNK = ROWS_PER_W // CHUNK
NBUF = 4
NOUTER = NCHUNK // NBUF
IDX_ROWS = ROWS_PER_W // IDX_MINOR
CHUNKS_PER_H = BATCH // CHUNK

_mesh = plsc.VectorSubcoreMesh(core_axis_name="c", subcore_axis_name="s")


@functools.partial(
    pl.kernel,
    mesh=_mesh,
    out_type=jax.ShapeDtypeStruct((HIST, BATCH, DIM), jnp.float32),
    scratch_types=[
        pltpu.VMEM((IDX_ROWS, IDX_MINOR), jnp.int32),
        pltpu.VMEM((CHUNK, DIM), jnp.float32),
        pltpu.VMEM((CHUNK, DIM), jnp.float32),
        pltpu.VMEM((CHUNK, DIM), jnp.float32),
        pltpu.VMEM((CHUNK, DIM), jnp.float32),
        pltpu.SemaphoreType.DMA,
        pltpu.SemaphoreType.DMA,
        pltpu.SemaphoreType.DMA,
        pltpu.SemaphoreType.DMA,
        pltpu.SemaphoreType.DMA,
        pltpu.SemaphoreType.DMA,
        pltpu.SemaphoreType.DMA,
        pltpu.SemaphoreType.DMA,
    ],
    compiler_params=pltpu.CompilerParams(use_tc_tiling_on_sc=False),
)
def _embed_sc(idx_hbm, table_hbm, out_hbm,
              idx_v, r0, r1, r2, r3,
              g0, g1, g2, g3, o0, o1, o2, o3):
    rows = (r0, r1, r2, r3)
    gsem = (g0, g1, g2, g3)
    osem = (o0, o1, o2, o3)

    wid = lax.axis_index("s") * NUM_CORES + lax.axis_index("c")
    base_chunk = wid * NCHUNK
    base_blk = wid * IDX_ROWS

    pltpu.sync_copy(idx_hbm.at[pl.ds(base_blk, IDX_ROWS)], idx_v)

    def issue_gather(c, slot):
        for jj in range(SUB):
            pltpu.async_copy(
                table_hbm.at[idx_v.at[c * SUB + jj]],
                rows[slot].at[pl.ds(jj * IDX_MINOR, IDX_MINOR)],
                gsem[slot],
            )

    def wait_gather(slot):
        pltpu.make_async_copy(
            out_hbm.at[0, pl.ds(0, CHUNK)], rows[slot], gsem[slot]
        ).wait()

    def issue_store(c, slot):
        gc = base_chunk + c
        h = gc // CHUNKS_PER_H
        b0 = (gc % CHUNKS_PER_H) * CHUNK
        pltpu.async_copy(
            rows[slot], out_hbm.at[h, pl.ds(b0, CHUNK)],
            osem[slot],
        )

    def wait_store(slot):
        pltpu.make_async_copy(
            rows[slot], out_hbm.at[0, pl.ds(0, CHUNK)], osem[slot]
        ).wait()

    issue_gather(0, 0)
    issue_gather(1, 1)
    wait_gather(0); issue_store(0, 0); issue_gather(2, 2)
    wait_gather(1); issue_store(1, 1); issue_gather(3, 3)
    wait_gather(2); issue_store(2, 2); wait_store(0); issue_gather(4, 0)
    wait_gather(3); issue_store(3, 3); wait_store(1); issue_gather(5, 1)

    def body(t, carry):
        c0 = t * NBUF
        for j in range(NBUF):
            c = c0 + j
            wait_gather(j)
            issue_store(c, j)
            wait_store((j + 2) % NBUF)
            issue_gather(c + 2, (j + 2) % NBUF)
        return carry

    lax.fori_loop(1, NOUTER - 1, body, 0)

    cL = (NOUTER - 1) * NBUF
    wait_gather(0); issue_store(cL + 0, 0); wait_store(2); issue_gather(cL + 2, 2)
    wait_gather(1); issue_store(cL + 1, 1); wait_store(3); issue_gather(cL + 3, 3)
    wait_gather(2); issue_store(cL + 2, 2); wait_store(0)
    wait_gather(3); issue_store(cL + 3, 3); wait_store(1)
    wait_store(2)
    wait_store(3)


def kernel(token_ids, weight):
    idx = token_ids.T.reshape(TOTAL // IDX_MINOR, IDX_MINOR)
    out = _embed_sc(idx, weight)
    return out.transpose(1, 0, 2)
